# baseline (speedup 1.0000x reference)
import functools

import jax
import jax.numpy as jnp
from jax import lax
from jax.experimental import pallas as pl
from jax.experimental.pallas import tpu as pltpu
from jax.experimental.pallas import tpu_sc as plsc

L = 16
NC = 2
NS = 16
NW = NC * NS
C = 1024


def kernel(x, edge_index, edge_attr, m1_w1, m1_b1, m1_w2, m1_b2, m1_w3, m1_b3,
           m2_w1, m2_b1, m2_w2, m2_b2, m2_w3, m2_b3):
    f32 = jnp.float32
    N = x.shape[0]
    E = edge_index.shape[1]

    nch = -(-E // (NW * C))
    nch += nch % 2
    Ep = nch * NW * C
    EPT = nch * C
    nrpt = -(-N // (NW * 128))
    Np = nrpt * NW * 128
    SL = Np // NS
    NR = Np // 128

    src = jnp.pad(edge_index[0], (0, Ep - E))
    nrpt_pad = N + (jnp.arange(Ep - E, dtype=jnp.int32) % (Np - N))
    dst = jnp.concatenate([edge_index[1], nrpt_pad])
    ea = jnp.pad(edge_attr[:, 0], (0, Ep - E))
    zeros_h = jnp.zeros((Np,), f32)
    dst2 = dst.reshape(Ep // 128, 128)
    x0 = jnp.pad(x[:, 0], (0, Np - N))
    x1 = jnp.pad(x[:, 1], (0, Np - N))

    A = m1_w1 @ m1_w2 @ m1_w3
    cvec = (m1_b1 @ m1_w2 + m1_b2) @ m1_w3 + m1_b3
    Bf = m2_w1 @ m2_w2 @ m2_w3
    dsc = (m2_b1 @ m2_w2 + m2_b2) @ m2_w3 + m2_b3
    W5 = jnp.concatenate([A, cvec[None, :], Bf[2:, 0][None, :]], axis=0)
    Wb = jnp.broadcast_to(W5.reshape(5, 32, 1), (5, 32, L)).reshape(160 * L)
    scal = jnp.pad(jnp.stack([Bf[0, 0], Bf[1, 0], dsc[0]]), (0, 13))

    mesh = plsc.VectorSubcoreMesh(core_axis_name="c", subcore_axis_name="s")
    cparams = pltpu.CompilerParams(needs_layout_passes=False)

    @functools.partial(
        pl.kernel, mesh=mesh, compiler_params=cparams,
        out_type=[jax.ShapeDtypeStruct((Ep,), f32),
                  jax.ShapeDtypeStruct((2 * Np,), f32)],
        scratch_types=[
            pltpu.VMEM((Np,), f32),
            pltpu.VMEM((C,), jnp.int32),
            pltpu.VMEM((C // 128, 128), jnp.int32),
            pltpu.VMEM((C,), f32),
            pltpu.VMEM((128,), f32),
            pltpu.VMEM_SHARED((Np,), f32),
            pltpu.SemaphoreType.DMA,
            pltpu.SemaphoreType.DMA,
        ],
    )
    def k_pre(x1_h, src_h, dst2_h, z_h, g1_h, cnt_h, x1t, srcv, dstv, g1v, ob,
              acc, dsem, ssem):
        cid = lax.axis_index("c")
        sid = lax.axis_index("s")
        wid = cid * NS + sid
        pltpu.sync_copy(x1_h, x1t)
        o16 = jnp.full((L,), 1.0, f32)
        for gg in range(128 // L):
            ob[pl.ds(gg * L, L)] = o16
        pltpu.sync_copy(z_h.at[pl.ds(sid * SL, SL)],
                        acc.at[pl.ds(sid * SL, SL)])
        plsc.subcore_barrier()

        def fchunk(ci, _):
            eb = pl.multiple_of((wid * nch + ci) * C, C)
            rb = pl.multiple_of((wid * nch + ci) * (C // 128), 8)
            hs = [pltpu.async_copy(src_h.at[pl.ds(eb, C)], srcv, dsem),
                  pltpu.async_copy(dst2_h.at[pl.ds(rb, C // 128)], dstv, dsem)]
            for h in hs:
                h.wait()

            @plsc.parallel_loop(0, C // L, 1, unroll=4)
            def fg(g):
                sl = pl.ds(g * L, L)
                g1v[sl] = plsc.load_gather(x1t, [srcv[sl]])
            pltpu.sync_copy(g1v, g1_h.at[pl.ds(eb, C)])
            sc = [pltpu.async_copy(ob, acc.at[dstv.at[r]], ssem, add=True)
                  for r in range(C // 128)]
            for h in sc:
                h.wait()
            return 0
        lax.fori_loop(0, nch, fchunk, 0)
        plsc.subcore_barrier()
        pltpu.sync_copy(acc.at[pl.ds(sid * SL, SL)],
                        cnt_h.at[pl.ds(cid * Np + sid * SL, SL)])

    @functools.partial(
        pl.kernel, mesh=mesh, compiler_params=cparams,
        out_type=jax.ShapeDtypeStruct((2 * Np,), f32),
        scratch_types=[
            pltpu.VMEM((Np,), f32),
            pltpu.VMEM((2, C), jnp.int32),
            pltpu.VMEM((2, C // 128, 128), jnp.int32),
            pltpu.VMEM((2, C), f32),
            pltpu.VMEM((2, C), f32),
            pltpu.VMEM((2, C), f32),
            pltpu.VMEM((160 * L,), f32),
            pltpu.VMEM_SHARED((Np,), f32),
            pltpu.SemaphoreType.DMA,
            pltpu.SemaphoreType.DMA,
        ],
    )
    def k_edge(x0_h, src_h, dst2_h, g1_h, ea_h, w_h, z_h, s_h,
               x0t, srcv, dstv, g1v, eav, sv, wb, acc, dsem, ssem):
        cid = lax.axis_index("c")
        sid = lax.axis_index("s")
        wid = cid * NS + sid
        pltpu.sync_copy(x0_h, x0t)
        pltpu.sync_copy(w_h, wb)
        pltpu.sync_copy(z_h.at[pl.ds(sid * SL, SL)],
                        acc.at[pl.ds(sid * SL, SL)])
        plsc.subcore_barrier()

        def in_copies(ci, b):
            eb = pl.multiple_of(wid * EPT + ci * C, C)
            rb = pl.multiple_of(wid * (EPT // 128) + ci * (C // 128), 8)
            return [pltpu.make_async_copy(src_h.at[pl.ds(eb, C)], srcv.at[b], dsem),
                    pltpu.make_async_copy(dst2_h.at[pl.ds(rb, C // 128)], dstv.at[b], dsem),
                    pltpu.make_async_copy(g1_h.at[pl.ds(eb, C)], g1v.at[b], dsem),
                    pltpu.make_async_copy(ea_h.at[pl.ds(eb, C)], eav.at[b], dsem)]

        def sc_copies(b):
            return [pltpu.make_async_copy(sv.at[b, pl.ds(r * 128, 128)],
                                          acc.at[dstv.at[b, r]], ssem)
                    for r in range(C // 128)]

        for h in in_copies(0, 0):
            h.start()

        def outer(it, _):
            for b in (0, 1):
                ci = it * 2 + b
                for h in in_copies(ci, b):
                    h.wait()

                for kb in range(4):
                    ws = [[wb[pl.ds((r * 32 + kb * 8 + j) * L, L)]
                           for j in range(8)] for r in range(5)]

                    @plsc.parallel_loop(0, C // L, 1, unroll=4)
                    def fg(g, kb=kb, ws=ws):
                        sl = pl.ds(g * L, L)
                        t = plsc.load_gather(x0t, [srcv[b, sl]])
                        g1 = g1v[b, sl]
                        e16 = eav[b, sl]
                        av = [jnp.zeros((L,), f32) for _ in range(2)]
                        for j in range(8):
                            u = (t * ws[0][j] + g1 * ws[1][j]
                                 + e16 * ws[2][j] + ws[3][j])
                            av[j % 2] = (av[j % 2]
                                         + ws[4][j] * jnp.maximum(u, 0.0))
                        part = av[0] + av[1]
                        if kb == 0:
                            sv[b, sl] = part
                        else:
                            sv[b, sl] = sv[b, sl] + part

                @pl.when(ci > 0)
                def _():
                    for h in sc_copies(1 - b):
                        h.wait()

                @pl.when(ci + 1 < nch)
                def _():
                    for h in in_copies(ci + 1, 1 - b):
                        h.start()
                for h in sc_copies(b):
                    h.start(add=True)
            return 0
        lax.fori_loop(0, nch // 2, outer, 0)
        for h in sc_copies(1):
            h.wait()
        plsc.subcore_barrier()
        pltpu.sync_copy(acc.at[pl.ds(sid * SL, SL)],
                        s_h.at[pl.ds(cid * Np + sid * SL, SL)])

    NPT = Np // NW

    @functools.partial(
        pl.kernel, mesh=mesh, compiler_params=cparams,
        out_type=jax.ShapeDtypeStruct((Np,), f32),
        scratch_types=[
            pltpu.VMEM((NPT,), f32),
            pltpu.VMEM((NPT,), f32),
            pltpu.VMEM((NPT,), f32),
            pltpu.VMEM((NPT,), f32),
            pltpu.VMEM((NPT,), f32),
            pltpu.VMEM((NPT,), f32),
            pltpu.VMEM((NPT,), f32),
            pltpu.VMEM((16,), f32),
        ],
    )
    def k_node(x0_h, x1_h, s_h, c_h, sc_h, o_h,
               x0v, x1v, s0v, s1v, c0v, c1v, ov, scv):
        cid = lax.axis_index("c")
        sid = lax.axis_index("s")
        wid = cid * NS + sid
        rb = pl.multiple_of(wid * NPT, 128)
        pltpu.sync_copy(x0_h.at[pl.ds(rb, NPT)], x0v)
        pltpu.sync_copy(x1_h.at[pl.ds(rb, NPT)], x1v)
        pltpu.sync_copy(s_h.at[pl.ds(rb, NPT)], s0v)
        pltpu.sync_copy(s_h.at[pl.ds(Np + rb, NPT)], s1v)
        pltpu.sync_copy(c_h.at[pl.ds(rb, NPT)], c0v)
        pltpu.sync_copy(c_h.at[pl.ds(Np + rb, NPT)], c1v)
        pltpu.sync_copy(sc_h, scv)
        s16 = scv[pl.ds(0, L)]
        B0 = s16[0]
        B1 = s16[1]
        dd = s16[2]

        def fr(g, _):
            sl = pl.ds(g * L, L)
            z = x0v[sl] * B0 + x1v[sl] * B1
            ssum = s0v[sl] + s1v[sl]
            csum = c0v[sl] + c1v[sl]
            z = z + ssum / jnp.maximum(csum, 1.0) + dd
            ov[sl] = 1.0 / (1.0 + jnp.exp(-z))
            return 0
        lax.fori_loop(0, NPT // L, fr, 0)
        pltpu.sync_copy(ov, o_h.at[pl.ds(rb, NPT)])

    g1e, cnt = k_pre(x1, src, dst2, zeros_h)
    x0c = x0
    for _ in range(3):
        S = k_edge(x0c, src, dst2, g1e, ea, Wb, zeros_h)
        x0c = k_node(x0c, x1, S, cnt, scal)
    return jnp.stack([x0c[:N], x[:, 1]], axis=1)


# speedup vs baseline: 1.0573x; 1.0573x over previous
import functools

import jax
import jax.numpy as jnp
from jax import lax
from jax.experimental import pallas as pl
from jax.experimental.pallas import tpu as pltpu
from jax.experimental.pallas import tpu_sc as plsc

L = 16
NC = 2
NS = 16
NW = NC * NS
C = 1024


def kernel(x, edge_index, edge_attr, m1_w1, m1_b1, m1_w2, m1_b2, m1_w3, m1_b3,
           m2_w1, m2_b1, m2_w2, m2_b2, m2_w3, m2_b3):
    f32 = jnp.float32
    N = x.shape[0]
    E = edge_index.shape[1]

    nch = -(-E // (NW * C))
    nch += nch % 2
    Ep = nch * NW * C
    EPT = nch * C
    nrpt = -(-N // (NW * 128))
    Np = nrpt * NW * 128
    SL = Np // NS
    NR = Np // 128

    src = jnp.pad(edge_index[0], (0, Ep - E))
    nrpt_pad = N + (jnp.arange(Ep - E, dtype=jnp.int32) % (Np - N))
    dst = jnp.concatenate([edge_index[1], nrpt_pad])
    ea = jnp.pad(edge_attr[:, 0], (0, Ep - E))
    zeros_h = jnp.zeros((Np,), f32)
    dst2 = dst.reshape(Ep // 128, 128)
    x0 = jnp.pad(x[:, 0], (0, Np - N))
    x1 = jnp.pad(x[:, 1], (0, Np - N))

    A = m1_w1 @ m1_w2 @ m1_w3
    cvec = (m1_b1 @ m1_w2 + m1_b2) @ m1_w3 + m1_b3
    Bf = m2_w1 @ m2_w2 @ m2_w3
    dsc = (m2_b1 @ m2_w2 + m2_b2) @ m2_w3 + m2_b3
    W5 = jnp.concatenate([A, cvec[None, :], Bf[2:, 0][None, :]], axis=0)
    Wb = jnp.broadcast_to(W5.reshape(5, 32, 1), (5, 32, L)).reshape(160 * L)
    scal = jnp.pad(jnp.stack([Bf[0, 0], Bf[1, 0], dsc[0]]), (0, 13))

    mesh = plsc.VectorSubcoreMesh(core_axis_name="c", subcore_axis_name="s")
    cparams = pltpu.CompilerParams(needs_layout_passes=False)

    @functools.partial(
        pl.kernel, mesh=mesh, compiler_params=cparams,
        out_type=[jax.ShapeDtypeStruct((Ep,), f32),
                  jax.ShapeDtypeStruct((2 * Np,), f32)],
        scratch_types=[
            pltpu.VMEM((Np,), f32),
            pltpu.VMEM((C,), jnp.int32),
            pltpu.VMEM((C // 128, 128), jnp.int32),
            pltpu.VMEM((C,), f32),
            pltpu.VMEM((128,), f32),
            pltpu.VMEM_SHARED((Np,), f32),
            pltpu.SemaphoreType.DMA,
            pltpu.SemaphoreType.DMA,
        ],
    )
    def k_pre(x1_h, src_h, dst2_h, z_h, g1_h, cnt_h, x1t, srcv, dstv, g1v, ob,
              acc, dsem, ssem):
        cid = lax.axis_index("c")
        sid = lax.axis_index("s")
        wid = cid * NS + sid
        pltpu.sync_copy(x1_h, x1t)
        o16 = jnp.full((L,), 1.0, f32)
        for gg in range(128 // L):
            ob[pl.ds(gg * L, L)] = o16
        pltpu.sync_copy(z_h.at[pl.ds(sid * SL, SL)],
                        acc.at[pl.ds(sid * SL, SL)])
        plsc.subcore_barrier()

        def fchunk(ci, _):
            eb = pl.multiple_of((wid * nch + ci) * C, C)
            rb = pl.multiple_of((wid * nch + ci) * (C // 128), 8)
            hs = [pltpu.async_copy(src_h.at[pl.ds(eb, C)], srcv, dsem),
                  pltpu.async_copy(dst2_h.at[pl.ds(rb, C // 128)], dstv, dsem)]
            for h in hs:
                h.wait()

            @plsc.parallel_loop(0, C // L, 1, unroll=2)
            def fg(g):
                sl = pl.ds(g * L, L)
                g1v[sl] = plsc.load_gather(x1t, [srcv[sl]])
            pltpu.sync_copy(g1v, g1_h.at[pl.ds(eb, C)])
            sc = [pltpu.async_copy(ob, acc.at[dstv.at[r]], ssem, add=True)
                  for r in range(C // 128)]
            for h in sc:
                h.wait()
            return 0
        lax.fori_loop(0, nch, fchunk, 0)
        plsc.subcore_barrier()
        pltpu.sync_copy(acc.at[pl.ds(sid * SL, SL)],
                        cnt_h.at[pl.ds(cid * Np + sid * SL, SL)])

    @functools.partial(
        pl.kernel, mesh=mesh, compiler_params=cparams,
        out_type=jax.ShapeDtypeStruct((2 * Np,), f32),
        scratch_types=[
            pltpu.VMEM((Np,), f32),
            pltpu.VMEM((2, C), jnp.int32),
            pltpu.VMEM((2, C // 128, 128), jnp.int32),
            pltpu.VMEM((2, C), f32),
            pltpu.VMEM((2, C), f32),
            pltpu.VMEM((2, C), f32),
            pltpu.VMEM((160 * L,), f32),
            pltpu.VMEM_SHARED((Np,), f32),
            pltpu.SemaphoreType.DMA,
            pltpu.SemaphoreType.DMA,
        ],
    )
    def k_edge(x0_h, src_h, dst2_h, g1_h, ea_h, w_h, z_h, s_h,
               x0t, srcv, dstv, g1v, eav, sv, wb, acc, dsem, ssem):
        cid = lax.axis_index("c")
        sid = lax.axis_index("s")
        wid = cid * NS + sid
        pltpu.sync_copy(x0_h, x0t)
        pltpu.sync_copy(w_h, wb)
        pltpu.sync_copy(z_h.at[pl.ds(sid * SL, SL)],
                        acc.at[pl.ds(sid * SL, SL)])
        plsc.subcore_barrier()

        def in_copies(ci, b):
            eb = pl.multiple_of(wid * EPT + ci * C, C)
            rb = pl.multiple_of(wid * (EPT // 128) + ci * (C // 128), 8)
            return [pltpu.make_async_copy(src_h.at[pl.ds(eb, C)], srcv.at[b], dsem),
                    pltpu.make_async_copy(dst2_h.at[pl.ds(rb, C // 128)], dstv.at[b], dsem),
                    pltpu.make_async_copy(g1_h.at[pl.ds(eb, C)], g1v.at[b], dsem),
                    pltpu.make_async_copy(ea_h.at[pl.ds(eb, C)], eav.at[b], dsem)]

        def sc_copies(b):
            return [pltpu.make_async_copy(sv.at[b, pl.ds(r * 128, 128)],
                                          acc.at[dstv.at[b, r]], ssem)
                    for r in range(C // 128)]

        for h in in_copies(0, 0):
            h.start()

        def outer(it, _):
            for b in (0, 1):
                ci = it * 2 + b
                for h in in_copies(ci, b):
                    h.wait()

                for kb in range(4):
                    ws = [[wb[pl.ds((r * 32 + kb * 8 + j) * L, L)]
                           for j in range(8)] for r in range(5)]

                    @plsc.parallel_loop(0, C // L, 1, unroll=2)
                    def fg(g, kb=kb, ws=ws):
                        sl = pl.ds(g * L, L)
                        t = plsc.load_gather(x0t, [srcv[b, sl]])
                        g1 = g1v[b, sl]
                        e16 = eav[b, sl]
                        av = [jnp.zeros((L,), f32) for _ in range(2)]
                        for j in range(8):
                            u = (t * ws[0][j] + g1 * ws[1][j]
                                 + e16 * ws[2][j] + ws[3][j])
                            av[j % 2] = (av[j % 2]
                                         + ws[4][j] * jnp.maximum(u, 0.0))
                        part = av[0] + av[1]
                        if kb == 0:
                            sv[b, sl] = part
                        else:
                            sv[b, sl] = sv[b, sl] + part

                @pl.when(ci > 0)
                def _():
                    for h in sc_copies(1 - b):
                        h.wait()

                @pl.when(ci + 1 < nch)
                def _():
                    for h in in_copies(ci + 1, 1 - b):
                        h.start()
                for h in sc_copies(b):
                    h.start(add=True)
            return 0
        lax.fori_loop(0, nch // 2, outer, 0)
        for h in sc_copies(1):
            h.wait()
        plsc.subcore_barrier()
        pltpu.sync_copy(acc.at[pl.ds(sid * SL, SL)],
                        s_h.at[pl.ds(cid * Np + sid * SL, SL)])

    NPT = Np // NW

    @functools.partial(
        pl.kernel, mesh=mesh, compiler_params=cparams,
        out_type=jax.ShapeDtypeStruct((Np,), f32),
        scratch_types=[
            pltpu.VMEM((NPT,), f32),
            pltpu.VMEM((NPT,), f32),
            pltpu.VMEM((NPT,), f32),
            pltpu.VMEM((NPT,), f32),
            pltpu.VMEM((NPT,), f32),
            pltpu.VMEM((NPT,), f32),
            pltpu.VMEM((NPT,), f32),
            pltpu.VMEM((16,), f32),
        ],
    )
    def k_node(x0_h, x1_h, s_h, c_h, sc_h, o_h,
               x0v, x1v, s0v, s1v, c0v, c1v, ov, scv):
        cid = lax.axis_index("c")
        sid = lax.axis_index("s")
        wid = cid * NS + sid
        rb = pl.multiple_of(wid * NPT, 128)
        pltpu.sync_copy(x0_h.at[pl.ds(rb, NPT)], x0v)
        pltpu.sync_copy(x1_h.at[pl.ds(rb, NPT)], x1v)
        pltpu.sync_copy(s_h.at[pl.ds(rb, NPT)], s0v)
        pltpu.sync_copy(s_h.at[pl.ds(Np + rb, NPT)], s1v)
        pltpu.sync_copy(c_h.at[pl.ds(rb, NPT)], c0v)
        pltpu.sync_copy(c_h.at[pl.ds(Np + rb, NPT)], c1v)
        pltpu.sync_copy(sc_h, scv)
        s16 = scv[pl.ds(0, L)]
        B0 = s16[0]
        B1 = s16[1]
        dd = s16[2]

        def fr(g, _):
            sl = pl.ds(g * L, L)
            z = x0v[sl] * B0 + x1v[sl] * B1
            ssum = s0v[sl] + s1v[sl]
            csum = c0v[sl] + c1v[sl]
            z = z + ssum / jnp.maximum(csum, 1.0) + dd
            ov[sl] = 1.0 / (1.0 + jnp.exp(-z))
            return 0
        lax.fori_loop(0, NPT // L, fr, 0)
        pltpu.sync_copy(ov, o_h.at[pl.ds(rb, NPT)])

    g1e, cnt = k_pre(x1, src, dst2, zeros_h)
    x0c = x0
    for _ in range(3):
        S = k_edge(x0c, src, dst2, g1e, ea, Wb, zeros_h)
        x0c = k_node(x0c, x1, S, cnt, scal)
    return jnp.stack([x0c[:N], x[:, 1]], axis=1)
